# Initial kernel scaffold; baseline (speedup 1.0000x reference)
import jax
import jax.numpy as jnp
from jax.experimental import pallas as pl


def kernel(q, q_val, w1, b1, w2, b2, w3, b3, w4, b4, w5, b5):
    raise NotImplementedError("write your pallas kernel here")


# speedup vs baseline: 5.0307x; 5.0307x over previous
import functools

import jax
import jax.numpy as jnp
from jax.experimental import pallas as pl
from jax.experimental.pallas import tpu as pltpu

B = 16
NPTS = 1920
GRID = 128
EPS_IN = 1e-5
W7 = (1/27., 3/27., 6/27., 7/27., 6/27., 3/27., 1/27.)


def _k1_body(q_ref, w_ref, b_ref, o_ref):
    o_ref[...] = jnp.dot(q_ref[...], w_ref[...],
                         preferred_element_type=jnp.float32) + b_ref[...]


def _inorm_relu(x):
    m = jnp.mean(x, axis=-1, keepdims=True)
    xm = x - m
    v = jnp.mean(xm * xm, axis=-1, keepdims=True)
    xn = xm * jax.lax.rsqrt(v + EPS_IN)
    return jnp.maximum(xn, 0.0)


def _kmid_body(x_ref, w_ref, b_ref, o_ref):
    x = _inorm_relu(x_ref[...])
    o_ref[...] = (x[:, :, None, :] * w_ref[...][None, :, :, None]
                  + b_ref[...][None, :, :, None])


def _k5_body(x_ref, w_ref, b_ref, o_ref):
    x = _inorm_relu(x_ref[0])
    q3 = jnp.dot(w_ref[...], x, preferred_element_type=jnp.float32) + b_ref[...]
    o_ref[0] = jnp.tanh(q3)


def _taps(p):
    x0f = jnp.floor(p)
    f = p - x0f
    x0 = x0f.astype(jnp.int32)
    m0 = (x0 >= 0) & (x0 <= GRID - 1)
    m1 = (x0 >= -1) & (x0 <= GRID - 2)
    a0 = jnp.where(m0, 1.0 - f, 0.0)
    a1 = jnp.where(m1, f, 0.0)
    ks = []
    for t in range(8):
        k = a0 * W7[t] if t < 7 else jnp.zeros_like(a0)
        if t >= 1:
            k = k + a1 * W7[t - 1]
        ks.append(k)
    return x0, ks


def _to_vox(g):
    return (g + 1.0) * 64.0 - 0.5


def _c1_body(c_ref, qv_ref, w_ref, pz_ref):
    iota = jax.lax.broadcasted_iota(jnp.int32, (GRID, NPTS), 0)

    def prof(row):
        x0, ks = _taps(_to_vox(row))
        p = jnp.zeros((GRID, NPTS), jnp.float32)
        for t in range(8):
            p = jnp.where(iota == (x0 + (t - 3)), ks[t], p)
        return p

    py = prof(c_ref[0, 1:2, :])
    pz = prof(c_ref[0, 2:3, :])
    val = jax.nn.sigmoid(qv_ref[...])
    w_ref[0] = jnp.concatenate([py * val, py], axis=0)
    pz_ref[0] = pz


def _c2_body(ct_ref, px_ref):
    iota = jax.lax.broadcasted_iota(jnp.int32, (NPTS, GRID), 1)
    x0, ks = _taps(_to_vox(ct_ref[0, :, 0:1]))
    p = jnp.zeros((NPTS, GRID), jnp.float32)
    for t in range(8):
        p = jnp.where(iota == (x0 + (t - 3)), ks[t], p)
    px_ref[0] = p


def _b_body(w_ref, px_ref, pzt_ref, o_ref):
    wm = w_ref[0]
    px = px_ref[0]
    for j in range(8):
        pz_row = pzt_ref[0, j:j + 1, :]
        acc = jnp.dot(wm * pz_row, px,
                      preferred_element_type=jnp.float32)
        o_ref[0, 0, j, :, :] = acc[:GRID] / (acc[GRID:] + 0.001)


@jax.jit
def kernel(q, q_val, w1, b1, w2, b2, w3, b3, w4, b4, w5, b5):
    f32 = jnp.float32

    qt = q.reshape(B, 256, 8).transpose(0, 2, 1).reshape(B * 8, 256)
    a = pl.pallas_call(
        _k1_body,
        out_shape=jax.ShapeDtypeStruct((B * 8, 240), f32),
    )(qt, w1.T, b1.reshape(1, 240))
    x = a.reshape(B, 8, 240).transpose(0, 2, 1).reshape(B, 64, 30)

    def mid(xin, w, b, c, j, g):
        return pl.pallas_call(
            _kmid_body,
            out_shape=jax.ShapeDtypeStruct((B, c, j, g), f32),
        )(xin, w.reshape(c, j), b.reshape(c, j))

    x = mid(x, w2, b2, 64, 2, 30).reshape(B, 32, 120)
    x = mid(x, w3, b3, 32, 2, 120).reshape(B, 16, 480)
    x = mid(x, w4, b4, 16, 4, 480).reshape(B, 16, 1920)

    coords = pl.pallas_call(
        _k5_body,
        grid=(B,),
        in_specs=[
            pl.BlockSpec((1, 16, NPTS), lambda b: (b, 0, 0)),
            pl.BlockSpec((3, 16), lambda b: (0, 0)),
            pl.BlockSpec((3, 1), lambda b: (0, 0)),
        ],
        out_specs=pl.BlockSpec((1, 3, NPTS), lambda b: (b, 0, 0)),
        out_shape=jax.ShapeDtypeStruct((B, 3, NPTS), f32),
        compiler_params=pltpu.CompilerParams(
            dimension_semantics=("parallel",)),
    )(x, w5, b5.reshape(3, 1))

    coords_t = coords.transpose(0, 2, 1)
    qv = q_val.reshape(1, NPTS)

    wmat, pzt = pl.pallas_call(
        _c1_body,
        grid=(B,),
        in_specs=[
            pl.BlockSpec((1, 3, NPTS), lambda b: (b, 0, 0)),
            pl.BlockSpec((1, NPTS), lambda b: (0, 0)),
        ],
        out_specs=[
            pl.BlockSpec((1, 256, NPTS), lambda b: (b, 0, 0)),
            pl.BlockSpec((1, GRID, NPTS), lambda b: (b, 0, 0)),
        ],
        out_shape=[
            jax.ShapeDtypeStruct((B, 256, NPTS), f32),
            jax.ShapeDtypeStruct((B, GRID, NPTS), f32),
        ],
        compiler_params=pltpu.CompilerParams(
            dimension_semantics=("parallel",)),
    )(coords, qv)

    px = pl.pallas_call(
        _c2_body,
        grid=(B,),
        in_specs=[pl.BlockSpec((1, NPTS, 3), lambda b: (b, 0, 0))],
        out_specs=pl.BlockSpec((1, NPTS, GRID), lambda b: (b, 0, 0)),
        out_shape=jax.ShapeDtypeStruct((B, NPTS, GRID), f32),
        compiler_params=pltpu.CompilerParams(
            dimension_semantics=("parallel",)),
    )(coords_t)

    out = pl.pallas_call(
        _b_body,
        grid=(B, GRID // 8),
        in_specs=[
            pl.BlockSpec((1, 256, NPTS), lambda b, zb: (b, 0, 0)),
            pl.BlockSpec((1, NPTS, GRID), lambda b, zb: (b, 0, 0)),
            pl.BlockSpec((1, 8, NPTS), lambda b, zb: (b, zb, 0)),
        ],
        out_specs=pl.BlockSpec((1, 1, 8, GRID, GRID),
                               lambda b, zb: (b, 0, zb, 0, 0)),
        out_shape=jax.ShapeDtypeStruct((B, 1, GRID, GRID, GRID), f32),
        compiler_params=pltpu.CompilerParams(
            dimension_semantics=("parallel", "arbitrary"),
            vmem_limit_bytes=100 * 1024 * 1024),
    )(wmat, px, pzt)

    return out


# speedup vs baseline: 5.0844x; 1.0107x over previous
import functools

import jax
import jax.numpy as jnp
from jax.experimental import pallas as pl
from jax.experimental.pallas import tpu as pltpu

B = 16
NPTS = 1920
GRID = 128
EPS_IN = 1e-5
W7 = (1/27., 3/27., 6/27., 7/27., 6/27., 3/27., 1/27.)


def _k1_body(q_ref, w_ref, b_ref, o_ref):
    o_ref[...] = jnp.dot(q_ref[...], w_ref[...],
                         preferred_element_type=jnp.float32) + b_ref[...]


def _inorm_relu(x):
    m = jnp.mean(x, axis=-1, keepdims=True)
    xm = x - m
    v = jnp.mean(xm * xm, axis=-1, keepdims=True)
    xn = xm * jax.lax.rsqrt(v + EPS_IN)
    return jnp.maximum(xn, 0.0)


def _kmid_body(x_ref, w_ref, b_ref, o_ref):
    x = _inorm_relu(x_ref[...])
    o_ref[...] = (x[:, :, None, :] * w_ref[...][None, :, :, None]
                  + b_ref[...][None, :, :, None])


def _k5_body(x_ref, w_ref, b_ref, o_ref):
    x = _inorm_relu(x_ref[0])
    q3 = jnp.dot(w_ref[...], x, preferred_element_type=jnp.float32) + b_ref[...]
    o_ref[0] = jnp.tanh(q3)


def _taps(p):
    x0f = jnp.floor(p)
    f = p - x0f
    x0 = x0f.astype(jnp.int32)
    m0 = (x0 >= 0) & (x0 <= GRID - 1)
    m1 = (x0 >= -1) & (x0 <= GRID - 2)
    a0 = jnp.where(m0, 1.0 - f, 0.0)
    a1 = jnp.where(m1, f, 0.0)
    ks = []
    for t in range(8):
        k = a0 * W7[t] if t < 7 else jnp.zeros_like(a0)
        if t >= 1:
            k = k + a1 * W7[t - 1]
        ks.append(k)
    return x0, ks


def _to_vox(g):
    return (g + 1.0) * 64.0 - 0.5


def _c1_body(c_ref, qv_ref, w_ref, pz_ref):
    iota = jax.lax.broadcasted_iota(jnp.int32, (GRID, NPTS), 0)

    def prof(row):
        x0, ks = _taps(_to_vox(row))
        p = jnp.zeros((GRID, NPTS), jnp.float32)
        for t in range(8):
            p = jnp.where(iota == (x0 + (t - 3)), ks[t], p)
        return p

    py = prof(c_ref[0, 1:2, :])
    pz = prof(c_ref[0, 2:3, :])
    val = jax.nn.sigmoid(qv_ref[...])
    w_ref[0] = jnp.concatenate([py * val, py], axis=0).astype(jnp.bfloat16)
    pz_ref[0] = pz.astype(jnp.bfloat16)


def _c2_body(ct_ref, px_ref):
    iota = jax.lax.broadcasted_iota(jnp.int32, (NPTS, GRID), 1)
    x0, ks = _taps(_to_vox(ct_ref[0, :, 0:1]))
    p = jnp.zeros((NPTS, GRID), jnp.float32)
    for t in range(8):
        p = jnp.where(iota == (x0 + (t - 3)), ks[t], p)
    px_ref[0] = p.astype(jnp.bfloat16)


def _b_body(w_ref, px_ref, pzt_ref, o_ref):
    wm = w_ref[0]
    px = px_ref[0]
    for j in range(8):
        pz_row = pzt_ref[0, j:j + 1, :]
        acc = jnp.dot(wm * pz_row, px,
                      preferred_element_type=jnp.float32)
        o_ref[0, 0, j, :, :] = acc[:GRID] / (acc[GRID:] + 0.001)


@jax.jit
def kernel(q, q_val, w1, b1, w2, b2, w3, b3, w4, b4, w5, b5):
    f32 = jnp.float32

    qt = q.reshape(B, 256, 8).transpose(0, 2, 1).reshape(B * 8, 256)
    a = pl.pallas_call(
        _k1_body,
        out_shape=jax.ShapeDtypeStruct((B * 8, 240), f32),
    )(qt, w1.T, b1.reshape(1, 240))
    x = a.reshape(B, 8, 240).transpose(0, 2, 1).reshape(B, 64, 30)

    def mid(xin, w, b, c, j, g):
        return pl.pallas_call(
            _kmid_body,
            out_shape=jax.ShapeDtypeStruct((B, c, j, g), f32),
        )(xin, w.reshape(c, j), b.reshape(c, j))

    x = mid(x, w2, b2, 64, 2, 30).reshape(B, 32, 120)
    x = mid(x, w3, b3, 32, 2, 120).reshape(B, 16, 480)
    x = mid(x, w4, b4, 16, 4, 480).reshape(B, 16, 1920)

    coords = pl.pallas_call(
        _k5_body,
        grid=(B,),
        in_specs=[
            pl.BlockSpec((1, 16, NPTS), lambda b: (b, 0, 0)),
            pl.BlockSpec((3, 16), lambda b: (0, 0)),
            pl.BlockSpec((3, 1), lambda b: (0, 0)),
        ],
        out_specs=pl.BlockSpec((1, 3, NPTS), lambda b: (b, 0, 0)),
        out_shape=jax.ShapeDtypeStruct((B, 3, NPTS), f32),
        compiler_params=pltpu.CompilerParams(
            dimension_semantics=("parallel",)),
    )(x, w5, b5.reshape(3, 1))

    coords_t = coords.transpose(0, 2, 1)
    qv = q_val.reshape(1, NPTS)

    wmat, pzt = pl.pallas_call(
        _c1_body,
        grid=(B,),
        in_specs=[
            pl.BlockSpec((1, 3, NPTS), lambda b: (b, 0, 0)),
            pl.BlockSpec((1, NPTS), lambda b: (0, 0)),
        ],
        out_specs=[
            pl.BlockSpec((1, 256, NPTS), lambda b: (b, 0, 0)),
            pl.BlockSpec((1, GRID, NPTS), lambda b: (b, 0, 0)),
        ],
        out_shape=[
            jax.ShapeDtypeStruct((B, 256, NPTS), jnp.bfloat16),
            jax.ShapeDtypeStruct((B, GRID, NPTS), jnp.bfloat16),
        ],
        compiler_params=pltpu.CompilerParams(
            dimension_semantics=("parallel",)),
    )(coords, qv)

    px = pl.pallas_call(
        _c2_body,
        grid=(B,),
        in_specs=[pl.BlockSpec((1, NPTS, 3), lambda b: (b, 0, 0))],
        out_specs=pl.BlockSpec((1, NPTS, GRID), lambda b: (b, 0, 0)),
        out_shape=jax.ShapeDtypeStruct((B, NPTS, GRID), jnp.bfloat16),
        compiler_params=pltpu.CompilerParams(
            dimension_semantics=("parallel",)),
    )(coords_t)

    out = pl.pallas_call(
        _b_body,
        grid=(B, GRID // 8),
        in_specs=[
            pl.BlockSpec((1, 256, NPTS), lambda b, zb: (b, 0, 0)),
            pl.BlockSpec((1, NPTS, GRID), lambda b, zb: (b, 0, 0)),
            pl.BlockSpec((1, 8, NPTS), lambda b, zb: (b, zb, 0)),
        ],
        out_specs=pl.BlockSpec((1, 1, 8, GRID, GRID),
                               lambda b, zb: (b, 0, zb, 0, 0)),
        out_shape=jax.ShapeDtypeStruct((B, 1, GRID, GRID, GRID), f32),
        compiler_params=pltpu.CompilerParams(
            dimension_semantics=("parallel", "arbitrary"),
            vmem_limit_bytes=100 * 1024 * 1024),
    )(wmat, px, pzt)

    return out


# speedup vs baseline: 8.4756x; 1.6670x over previous
import functools

import jax
import jax.numpy as jnp
from jax.experimental import pallas as pl
from jax.experimental.pallas import tpu as pltpu

B = 16
NPTS = 1920
GRID = 128
EPS_IN = 1e-5
W7 = (1/27., 3/27., 6/27., 7/27., 6/27., 3/27., 1/27.)


def _k1_body(q_ref, w_ref, b_ref, o_ref):
    o_ref[...] = jnp.dot(q_ref[...], w_ref[...],
                         preferred_element_type=jnp.float32) + b_ref[...]


def _inorm_relu(x):
    m = jnp.mean(x, axis=-1, keepdims=True)
    xm = x - m
    v = jnp.mean(xm * xm, axis=-1, keepdims=True)
    xn = xm * jax.lax.rsqrt(v + EPS_IN)
    return jnp.maximum(xn, 0.0)


def _kmid_body(x_ref, w_ref, b_ref, o_ref):
    x = _inorm_relu(x_ref[...])
    o_ref[...] = (x[:, :, None, :] * w_ref[...][None, :, :, None]
                  + b_ref[...][None, :, :, None])


def _k5_body(x_ref, w_ref, b_ref, o_ref):
    x = _inorm_relu(x_ref[0])
    q3 = jnp.dot(w_ref[...], x, preferred_element_type=jnp.float32) + b_ref[...]
    o_ref[0] = jnp.tanh(q3)


def _taps(p):
    x0f = jnp.floor(p)
    f = p - x0f
    x0 = x0f.astype(jnp.int32)
    m0 = (x0 >= 0) & (x0 <= GRID - 1)
    m1 = (x0 >= -1) & (x0 <= GRID - 2)
    a0 = jnp.where(m0, 1.0 - f, 0.0)
    a1 = jnp.where(m1, f, 0.0)
    ks = []
    for t in range(8):
        k = a0 * W7[t] if t < 7 else jnp.zeros_like(a0)
        if t >= 1:
            k = k + a1 * W7[t - 1]
        ks.append(k)
    return x0, ks


def _to_vox(g):
    return (g + 1.0) * 64.0 - 0.5


def _c1_body(c_ref, qv_ref, w_ref, pz_ref, px_ref):
    iota = jax.lax.broadcasted_iota(jnp.int32, (GRID, NPTS), 0)

    def prof(row):
        x0, ks = _taps(_to_vox(row))
        p = jnp.zeros((GRID, NPTS), jnp.float32)
        for t in range(8):
            p = jnp.where(iota == (x0 + (t - 3)), ks[t], p)
        return p

    px = prof(c_ref[0, 0:1, :])
    py = prof(c_ref[0, 1:2, :])
    pz = prof(c_ref[0, 2:3, :])
    val = jax.nn.sigmoid(qv_ref[...])
    w_ref[0] = jnp.concatenate([py * val, py], axis=0).astype(jnp.bfloat16)
    pz_ref[0] = pz.astype(jnp.bfloat16)
    px_ref[0] = px.astype(jnp.bfloat16)


def _b_body(w_ref, pxt_ref, pzt_ref, o_ref):
    wm = w_ref[0]
    pxt = pxt_ref[0]
    for j in range(4):
        r0 = pxt * pzt_ref[0, 2 * j:2 * j + 1, :]
        r1 = pxt * pzt_ref[0, 2 * j + 1:2 * j + 2, :]
        rhs = jnp.concatenate([r0, r1], axis=0)
        acc = jax.lax.dot_general(
            wm, rhs, (((1,), (1,)), ((), ())),
            preferred_element_type=jnp.float32)
        a0 = acc[:, :GRID]
        a1 = acc[:, GRID:]
        o_ref[0, 0, 2 * j, :, :] = a0[:GRID] / (a0[GRID:] + 0.001)
        o_ref[0, 0, 2 * j + 1, :, :] = a1[:GRID] / (a1[GRID:] + 0.001)


@jax.jit
def kernel(q, q_val, w1, b1, w2, b2, w3, b3, w4, b4, w5, b5):
    f32 = jnp.float32

    qt = q.reshape(B, 256, 8).transpose(0, 2, 1).reshape(B * 8, 256)
    a = pl.pallas_call(
        _k1_body,
        out_shape=jax.ShapeDtypeStruct((B * 8, 240), f32),
    )(qt, w1.T, b1.reshape(1, 240))
    x = a.reshape(B, 8, 240).transpose(0, 2, 1).reshape(B, 64, 30)

    def mid(xin, w, b, c, j, g):
        return pl.pallas_call(
            _kmid_body,
            out_shape=jax.ShapeDtypeStruct((B, c, j, g), f32),
        )(xin, w.reshape(c, j), b.reshape(c, j))

    x = mid(x, w2, b2, 64, 2, 30).reshape(B, 32, 120)
    x = mid(x, w3, b3, 32, 2, 120).reshape(B, 16, 480)
    x = mid(x, w4, b4, 16, 4, 480).reshape(B, 16, 1920)

    coords = pl.pallas_call(
        _k5_body,
        grid=(B,),
        in_specs=[
            pl.BlockSpec((1, 16, NPTS), lambda b: (b, 0, 0)),
            pl.BlockSpec((3, 16), lambda b: (0, 0)),
            pl.BlockSpec((3, 1), lambda b: (0, 0)),
        ],
        out_specs=pl.BlockSpec((1, 3, NPTS), lambda b: (b, 0, 0)),
        out_shape=jax.ShapeDtypeStruct((B, 3, NPTS), f32),
        compiler_params=pltpu.CompilerParams(
            dimension_semantics=("parallel",)),
    )(x, w5, b5.reshape(3, 1))

    qv = q_val.reshape(1, NPTS)

    wmat, pzt, pxt = pl.pallas_call(
        _c1_body,
        grid=(B,),
        in_specs=[
            pl.BlockSpec((1, 3, NPTS), lambda b: (b, 0, 0)),
            pl.BlockSpec((1, NPTS), lambda b: (0, 0)),
        ],
        out_specs=[
            pl.BlockSpec((1, 256, NPTS), lambda b: (b, 0, 0)),
            pl.BlockSpec((1, GRID, NPTS), lambda b: (b, 0, 0)),
            pl.BlockSpec((1, GRID, NPTS), lambda b: (b, 0, 0)),
        ],
        out_shape=[
            jax.ShapeDtypeStruct((B, 256, NPTS), jnp.bfloat16),
            jax.ShapeDtypeStruct((B, GRID, NPTS), jnp.bfloat16),
            jax.ShapeDtypeStruct((B, GRID, NPTS), jnp.bfloat16),
        ],
        compiler_params=pltpu.CompilerParams(
            dimension_semantics=("parallel",)),
    )(coords, qv)

    out = pl.pallas_call(
        _b_body,
        grid=(B, GRID // 8),
        in_specs=[
            pl.BlockSpec((1, 256, NPTS), lambda b, zb: (b, 0, 0)),
            pl.BlockSpec((1, GRID, NPTS), lambda b, zb: (b, 0, 0)),
            pl.BlockSpec((1, 8, NPTS), lambda b, zb: (b, zb, 0)),
        ],
        out_specs=pl.BlockSpec((1, 1, 8, GRID, GRID),
                               lambda b, zb: (b, 0, zb, 0, 0)),
        out_shape=jax.ShapeDtypeStruct((B, 1, GRID, GRID, GRID), f32),
        compiler_params=pltpu.CompilerParams(
            dimension_semantics=("parallel", "arbitrary"),
            vmem_limit_bytes=100 * 1024 * 1024),
    )(wmat, pxt, pzt)

    return out


# speedup vs baseline: 9.8733x; 1.1649x over previous
import functools

import jax
import jax.numpy as jnp
from jax.experimental import pallas as pl
from jax.experimental.pallas import tpu as pltpu

B = 16
ZB = 64
NPTS = 1920
GRID = 128
EPS_IN = 1e-5
W7 = (1/27., 3/27., 6/27., 7/27., 6/27., 3/27., 1/27.)


def _k1_body(q_ref, w_ref, b_ref, o_ref):
    o_ref[...] = jnp.dot(q_ref[...], w_ref[...],
                         preferred_element_type=jnp.float32) + b_ref[...]


def _inorm_relu(x):
    m = jnp.mean(x, axis=-1, keepdims=True)
    xm = x - m
    v = jnp.mean(xm * xm, axis=-1, keepdims=True)
    xn = xm * jax.lax.rsqrt(v + EPS_IN)
    return jnp.maximum(xn, 0.0)


def _kmid_body(x_ref, w_ref, b_ref, o_ref):
    x = _inorm_relu(x_ref[...])
    o_ref[...] = (x[:, :, None, :] * w_ref[...][None, :, :, None]
                  + b_ref[...][None, :, :, None])


def _k5_body(x_ref, w_ref, b_ref, o_ref):
    x = _inorm_relu(x_ref[0])
    q3 = jnp.dot(w_ref[...], x, preferred_element_type=jnp.float32) + b_ref[...]
    o_ref[0] = jnp.tanh(q3)


def _taps(p):
    x0f = jnp.floor(p)
    f = p - x0f
    x0 = x0f.astype(jnp.int32)
    m0 = (x0 >= 0) & (x0 <= GRID - 1)
    m1 = (x0 >= -1) & (x0 <= GRID - 2)
    a0 = jnp.where(m0, 1.0 - f, 0.0)
    a1 = jnp.where(m1, f, 0.0)
    ks = []
    for t in range(8):
        k = a0 * W7[t] if t < 7 else jnp.zeros_like(a0)
        if t >= 1:
            k = k + a1 * W7[t - 1]
        ks.append(k)
    return x0, ks


def _to_vox(g):
    return (g + 1.0) * 64.0 - 0.5


def _c1_body(c_ref, qv_ref, w_ref, pz_ref, px_ref):
    iota = jax.lax.broadcasted_iota(jnp.int32, (GRID, NPTS), 0)

    def prof(row):
        x0, ks = _taps(_to_vox(row))
        p = jnp.zeros((GRID, NPTS), jnp.float32)
        for t in range(8):
            p = jnp.where(iota == (x0 + (t - 3)), ks[t], p)
        return p

    px = prof(c_ref[0, 0:1, :])
    py = prof(c_ref[0, 1:2, :])
    pz = prof(c_ref[0, 2:3, :])
    val = jax.nn.sigmoid(qv_ref[...])
    w_ref[0] = jnp.concatenate([py * val, py], axis=0).astype(jnp.bfloat16)
    pz_ref[0] = pz.astype(jnp.bfloat16)
    px_ref[0] = px.astype(jnp.bfloat16)


def _b_body(w_ref, pxt_ref, pzt_ref, o_ref):
    wm = w_ref[0]
    pxt = pxt_ref[0]
    zrows = 2
    for j in range(ZB // zrows):
        rhs = jnp.concatenate(
            [pxt * pzt_ref[0, j * zrows + i:j * zrows + i + 1, :]
             for i in range(zrows)], axis=0)
        acc = jax.lax.dot_general(
            wm, rhs, (((1,), (1,)), ((), ())),
            preferred_element_type=jnp.float32)
        for i in range(zrows):
            a = acc[:, i * GRID:(i + 1) * GRID]
            o_ref[0, 0, j * zrows + i, :, :] = a[:GRID] / (a[GRID:] + 0.001)


@jax.jit
def kernel(q, q_val, w1, b1, w2, b2, w3, b3, w4, b4, w5, b5):
    f32 = jnp.float32

    qt = q.reshape(B, 256, 8).transpose(0, 2, 1).reshape(B * 8, 256)
    a = pl.pallas_call(
        _k1_body,
        out_shape=jax.ShapeDtypeStruct((B * 8, 240), f32),
    )(qt, w1.T, b1.reshape(1, 240))
    x = a.reshape(B, 8, 240).transpose(0, 2, 1).reshape(B, 64, 30)

    def mid(xin, w, b, c, j, g):
        return pl.pallas_call(
            _kmid_body,
            out_shape=jax.ShapeDtypeStruct((B, c, j, g), f32),
        )(xin, w.reshape(c, j), b.reshape(c, j))

    x = mid(x, w2, b2, 64, 2, 30).reshape(B, 32, 120)
    x = mid(x, w3, b3, 32, 2, 120).reshape(B, 16, 480)
    x = mid(x, w4, b4, 16, 4, 480).reshape(B, 16, 1920)

    coords = pl.pallas_call(
        _k5_body,
        grid=(B,),
        in_specs=[
            pl.BlockSpec((1, 16, NPTS), lambda b: (b, 0, 0)),
            pl.BlockSpec((3, 16), lambda b: (0, 0)),
            pl.BlockSpec((3, 1), lambda b: (0, 0)),
        ],
        out_specs=pl.BlockSpec((1, 3, NPTS), lambda b: (b, 0, 0)),
        out_shape=jax.ShapeDtypeStruct((B, 3, NPTS), f32),
        compiler_params=pltpu.CompilerParams(
            dimension_semantics=("parallel",)),
    )(x, w5, b5.reshape(3, 1))

    qv = q_val.reshape(1, NPTS)

    wmat, pzt, pxt = pl.pallas_call(
        _c1_body,
        grid=(B,),
        in_specs=[
            pl.BlockSpec((1, 3, NPTS), lambda b: (b, 0, 0)),
            pl.BlockSpec((1, NPTS), lambda b: (0, 0)),
        ],
        out_specs=[
            pl.BlockSpec((1, 256, NPTS), lambda b: (b, 0, 0)),
            pl.BlockSpec((1, GRID, NPTS), lambda b: (b, 0, 0)),
            pl.BlockSpec((1, GRID, NPTS), lambda b: (b, 0, 0)),
        ],
        out_shape=[
            jax.ShapeDtypeStruct((B, 256, NPTS), jnp.bfloat16),
            jax.ShapeDtypeStruct((B, GRID, NPTS), jnp.bfloat16),
            jax.ShapeDtypeStruct((B, GRID, NPTS), jnp.bfloat16),
        ],
        compiler_params=pltpu.CompilerParams(
            dimension_semantics=("parallel",)),
    )(coords, qv)

    out = pl.pallas_call(
        _b_body,
        grid=(B, GRID // ZB),
        in_specs=[
            pl.BlockSpec((1, 256, NPTS), lambda b, zb: (b, 0, 0)),
            pl.BlockSpec((1, GRID, NPTS), lambda b, zb: (b, 0, 0)),
            pl.BlockSpec((1, ZB, NPTS), lambda b, zb: (b, zb, 0)),
        ],
        out_specs=pl.BlockSpec((1, 1, ZB, GRID, GRID),
                               lambda b, zb: (b, 0, zb, 0, 0)),
        out_shape=jax.ShapeDtypeStruct((B, 1, GRID, GRID, GRID), f32),
        compiler_params=pltpu.CompilerParams(
            dimension_semantics=("parallel", "arbitrary"),
            vmem_limit_bytes=100 * 1024 * 1024),
    )(wmat, pxt, pzt)

    return out


# speedup vs baseline: 10.4056x; 1.0539x over previous
import functools

import jax
import jax.numpy as jnp
from jax.experimental import pallas as pl
from jax.experimental.pallas import tpu as pltpu

B = 16
ZB = 64
NPTS = 1920
GRID = 128
EPS_IN = 1e-5
W7 = (1/27., 3/27., 6/27., 7/27., 6/27., 3/27., 1/27.)


def _k1_body(q_ref, w_ref, b_ref, o_ref):
    o_ref[...] = jnp.dot(q_ref[...], w_ref[...],
                         preferred_element_type=jnp.float32) + b_ref[...]


def _inorm_relu(x):
    m = jnp.mean(x, axis=-1, keepdims=True)
    xm = x - m
    v = jnp.mean(xm * xm, axis=-1, keepdims=True)
    xn = xm * jax.lax.rsqrt(v + EPS_IN)
    return jnp.maximum(xn, 0.0)


def _kmid_body(x_ref, w_ref, b_ref, o_ref):
    x = _inorm_relu(x_ref[...])
    o_ref[...] = (x[:, :, None, :] * w_ref[...][None, :, :, None]
                  + b_ref[...][None, :, :, None])


def _k5_body(x_ref, w_ref, b_ref, o_ref):
    x = _inorm_relu(x_ref[0])
    q3 = jnp.dot(w_ref[...], x, preferred_element_type=jnp.float32) + b_ref[...]
    o_ref[0] = jnp.tanh(q3)


def _taps(p):
    x0f = jnp.floor(p)
    f = p - x0f
    x0 = x0f.astype(jnp.int32)
    m0 = (x0 >= 0) & (x0 <= GRID - 1)
    m1 = (x0 >= -1) & (x0 <= GRID - 2)
    a0 = jnp.where(m0, 1.0 - f, 0.0)
    a1 = jnp.where(m1, f, 0.0)
    ks = []
    for t in range(8):
        k = a0 * W7[t] if t < 7 else jnp.zeros_like(a0)
        if t >= 1:
            k = k + a1 * W7[t - 1]
        ks.append(k)
    return x0, ks


def _to_vox(g):
    return (g + 1.0) * 64.0 - 0.5


def _b_body(c_ref, qv_ref, o_ref):
    iota = jax.lax.broadcasted_iota(jnp.int32, (GRID, NPTS), 0)

    def prof(row):
        x0, ks = _taps(_to_vox(row))
        p = jnp.zeros((GRID, NPTS), jnp.float32)
        for t in range(8):
            p = jnp.where(iota == (x0 + (t - 3)), ks[t], p)
        return p

    pxt = prof(c_ref[0, 0:1, :]).astype(jnp.bfloat16)
    py = prof(c_ref[0, 1:2, :])
    pzt = prof(c_ref[0, 2:3, :]).astype(jnp.bfloat16)
    val = jax.nn.sigmoid(qv_ref[...])
    wm = jnp.concatenate([py * val, py], axis=0).astype(jnp.bfloat16)

    for j in range(GRID // 2):
        rhs = jnp.concatenate(
            [pxt * pzt[2 * j + i:2 * j + i + 1, :]
             for i in range(2)], axis=0)
        acc = jax.lax.dot_general(
            wm, rhs, (((1,), (1,)), ((), ())),
            preferred_element_type=jnp.float32)
        for i in range(2):
            a = acc[:, i * GRID:(i + 1) * GRID]
            o_ref[0, 0, 2 * j + i, :, :] = a[:GRID] / (a[GRID:] + 0.001)


@jax.jit
def kernel(q, q_val, w1, b1, w2, b2, w3, b3, w4, b4, w5, b5):
    f32 = jnp.float32

    qt = q.reshape(B, 256, 8).transpose(0, 2, 1).reshape(B * 8, 256)
    a = pl.pallas_call(
        _k1_body,
        out_shape=jax.ShapeDtypeStruct((B * 8, 240), f32),
    )(qt, w1.T, b1.reshape(1, 240))
    x = a.reshape(B, 8, 240).transpose(0, 2, 1).reshape(B, 64, 30)

    def mid(xin, w, b, c, j, g):
        return pl.pallas_call(
            _kmid_body,
            out_shape=jax.ShapeDtypeStruct((B, c, j, g), f32),
        )(xin, w.reshape(c, j), b.reshape(c, j))

    x = mid(x, w2, b2, 64, 2, 30).reshape(B, 32, 120)
    x = mid(x, w3, b3, 32, 2, 120).reshape(B, 16, 480)
    x = mid(x, w4, b4, 16, 4, 480).reshape(B, 16, 1920)

    coords = pl.pallas_call(
        _k5_body,
        grid=(B,),
        in_specs=[
            pl.BlockSpec((1, 16, NPTS), lambda b: (b, 0, 0)),
            pl.BlockSpec((3, 16), lambda b: (0, 0)),
            pl.BlockSpec((3, 1), lambda b: (0, 0)),
        ],
        out_specs=pl.BlockSpec((1, 3, NPTS), lambda b: (b, 0, 0)),
        out_shape=jax.ShapeDtypeStruct((B, 3, NPTS), f32),
        compiler_params=pltpu.CompilerParams(
            dimension_semantics=("parallel",)),
    )(x, w5, b5.reshape(3, 1))

    qv = q_val.reshape(1, NPTS)

    out = pl.pallas_call(
        _b_body,
        grid=(B,),
        in_specs=[
            pl.BlockSpec((1, 3, NPTS), lambda b: (b, 0, 0)),
            pl.BlockSpec((1, NPTS), lambda b: (0, 0)),
        ],
        out_specs=pl.BlockSpec((1, 1, GRID, GRID, GRID),
                               lambda b: (b, 0, 0, 0, 0)),
        out_shape=jax.ShapeDtypeStruct((B, 1, GRID, GRID, GRID), f32),
        compiler_params=pltpu.CompilerParams(
            dimension_semantics=("parallel",),
            vmem_limit_bytes=100 * 1024 * 1024),
    )(coords, qv)

    return out
